# baseline (speedup 1.0000x reference)
import functools

import jax
import jax.numpy as jnp
from jax import lax
from jax.experimental import pallas as pl
from jax.experimental.pallas import tpu as pltpu
from jax.experimental.pallas import tpu_sc as plsc

_BLK = 256


def _leaky(v):
    return jnp.where(v >= 0, v, 0.01 * v)


def _encoder_call(x, w1, b1, w2, b2):
    n = x.shape[0]
    l = x.shape[1] - 1
    inter = w1.shape[0]
    enc = w2.shape[0]
    nb = n // _BLK

    def body(x_ref, w1_ref, b1_ref, w2_ref, b2_ref, z_ref, w1b, w2b):
        @pl.when(pl.program_id(0) == 0)
        def _():
            w1b[...] = w1_ref[...].astype(jnp.bfloat16)
            w2b[...] = w2_ref[...].astype(jnp.bfloat16)

        xb = x_ref[...].astype(jnp.bfloat16)
        h = lax.dot_general(xb, w1b[...], (((1,), (1,)), ((), ())),
                            preferred_element_type=jnp.float32)
        h = _leaky(h + b1_ref[...]).astype(jnp.bfloat16)
        z = lax.dot_general(h, w2b[...], (((1,), (1,)), ((), ())),
                            preferred_element_type=jnp.float32)
        z_ref[...] = _leaky(z + b2_ref[...])

    return pl.pallas_call(
        body,
        grid=(nb,),
        in_specs=[
            pl.BlockSpec((_BLK, l), lambda i: (i, 0)),
            pl.BlockSpec((inter, l), lambda i: (0, 0)),
            pl.BlockSpec((1, inter), lambda i: (0, 0)),
            pl.BlockSpec((enc, inter), lambda i: (0, 0)),
            pl.BlockSpec((1, enc), lambda i: (0, 0)),
        ],
        out_specs=pl.BlockSpec((_BLK, enc), lambda i: (i, 0)),
        out_shape=jax.ShapeDtypeStruct((n, enc), jnp.float32),
        scratch_shapes=[
            pltpu.VMEM((inter, l), jnp.bfloat16),
            pltpu.VMEM((enc, inter), jnp.bfloat16),
        ],
    )(x, w1, b1, w2, b2)


def _scatter_rows(z, inv):
    n, enc = z.shape
    info = plsc.get_sparse_core_info()
    nc = info.num_cores
    nw = nc * info.num_subcores
    rows_per_w = n // nw
    ch = min(rows_per_w, 64)
    nchunk = rows_per_w // ch
    mesh = plsc.VectorSubcoreMesh(core_axis_name="c", subcore_axis_name="s")

    @functools.partial(
        pl.kernel, mesh=mesh,
        out_type=jax.ShapeDtypeStruct((n, enc), jnp.float32),
        scratch_types=[
            pltpu.VMEM((ch,), jnp.int32),
            pltpu.VMEM((ch,), jnp.int32),
            pltpu.VMEM((ch, enc), jnp.float32),
            pltpu.VMEM((ch, enc), jnp.float32),
            pltpu.SemaphoreType.DMA,
            pltpu.SemaphoreType.DMA,
            pltpu.SemaphoreType.DMA,
        ],
    )
    def gk(z_hbm, inv_hbm, out_hbm, i0, i1, r0, r1, sl0, sl1, ss):
        idxb, rowb, slds = (i0, i1), (r0, r1), (sl0, sl1)
        wid = lax.axis_index("s") * nc + lax.axis_index("c")
        base = wid * rows_per_w

        def start(c):
            off = base + c * ch
            b = c % 2
            return (pltpu.async_copy(inv_hbm.at[pl.ds(off, ch)], idxb[b],
                                     slds[b]),
                    pltpu.async_copy(z_hbm.at[pl.ds(off, ch)], rowb[b],
                                     slds[b]))

        pend = start(0)
        scat = None
        for c in range(nchunk):
            b = c % 2
            for cp in pend:
                cp.wait()
            if scat is not None:
                scat.wait()
            if c + 1 < nchunk:
                pend = start(c + 1)
            scat = pltpu.async_copy(rowb[b], out_hbm.at[idxb[b]], ss)
        scat.wait()

    return gk(z, inv)


def _decoder_call(zs, w1, b1, w2, b2, sched, nsteps):
    n, enc = zs.shape
    inter = w1.shape[1]
    l = w2.shape[1]

    def body(sched_ref, zs_ref, w1_ref, b1_ref, w2_ref, b2_ref, out_ref):
        j = pl.program_id(0)
        lo = sched_ref[2, j]
        hi = sched_ref[3, j]
        h = lax.dot_general(zs_ref[...], w1_ref[0], (((1,), (1,)), ((), ())),
                            preferred_element_type=jnp.float32)
        h = _leaky(h + b1_ref[0])
        o = lax.dot_general(h, w2_ref[0], (((1,), (1,)), ((), ())),
                            preferred_element_type=jnp.float32)
        o = o + b2_ref[0]
        rows = lax.broadcasted_iota(jnp.int32, (_BLK, 1), 0)
        mask = (rows >= lo) & (rows < hi)
        out_ref[...] = jnp.where(mask, o, out_ref[...])

    grid_spec = pltpu.PrefetchScalarGridSpec(
        num_scalar_prefetch=1,
        grid=(nsteps,),
        in_specs=[
            pl.BlockSpec((_BLK, enc), lambda j, s: (s[1, j], 0)),
            pl.BlockSpec((1, inter, enc), lambda j, s: (s[0, j], 0, 0)),
            pl.BlockSpec((1, 1, inter), lambda j, s: (s[0, j], 0, 0)),
            pl.BlockSpec((1, l, inter), lambda j, s: (s[0, j], 0, 0)),
            pl.BlockSpec((1, 1, l), lambda j, s: (s[0, j], 0, 0)),
        ],
        out_specs=pl.BlockSpec((_BLK, l), lambda j, s: (s[1, j], 0)),
    )
    return pl.pallas_call(
        body,
        grid_spec=grid_spec,
        out_shape=jax.ShapeDtypeStruct((n, l), jnp.float32),
    )(sched, zs, w1, b1, w2, b2)


def _dispatch_plan(ids, e, n, nsteps):
    nb = n // _BLK
    oh = (ids[None, :] == jnp.arange(e, dtype=jnp.int32)[:, None])
    cum = jnp.cumsum(oh.astype(jnp.int32), axis=1)
    counts = cum[:, -1]
    seg_end = jnp.cumsum(counts)
    seg_start = seg_end - counts
    inv = jnp.sum(jnp.where(oh, cum + seg_start[:, None], 0), axis=0) - 1
    inv = inv.astype(jnp.int32)

    first_blk = seg_start // _BLK
    last_blk = jnp.where(counts > 0, (seg_end - 1) // _BLK, first_blk)
    steps_e = jnp.where(counts > 0, last_blk - first_blk + 1, 0)
    cum_steps = jnp.cumsum(steps_e)
    off_e = cum_steps - steps_e
    total = cum_steps[-1]

    jj = jnp.arange(nsteps, dtype=jnp.int32)
    e_j = jnp.sum(jj[:, None] >= cum_steps[None, :], axis=1).astype(jnp.int32)
    e_j = jnp.minimum(e_j, e - 1)
    blk_j = first_blk[e_j] + (jj - off_e[e_j])
    lo = jnp.maximum(seg_start[e_j] - blk_j * _BLK, 0)
    hi = jnp.minimum(seg_end[e_j] - blk_j * _BLK, _BLK)

    dummy = jj >= total
    e_last = jnp.max(jnp.where(counts > 0, jnp.arange(e, dtype=jnp.int32), -1))
    e_j = jnp.where(dummy, e_last, e_j)
    blk_j = jnp.where(dummy, nb - 1, blk_j)
    lo = jnp.where(dummy, 0, lo)
    hi = jnp.where(dummy, 0, hi)
    sched = jnp.stack([e_j, blk_j, lo, hi]).astype(jnp.int32)
    return inv, sched


def _sc_copy_probe(w):
    flat = w.reshape(-1)
    m = flat.shape[0]
    info = plsc.get_sparse_core_info()
    nc = info.num_cores
    nw = nc * info.num_subcores
    per_w = m // nw
    ch = 32768
    nchunk = per_w // ch
    mesh = plsc.VectorSubcoreMesh(core_axis_name="c", subcore_axis_name="s")

    @functools.partial(
        pl.kernel, mesh=mesh,
        out_type=jax.ShapeDtypeStruct((m,), jnp.float32),
        scratch_types=[pltpu.VMEM((ch,), jnp.float32)],
    )
    def ck(w_hbm, o_hbm, buf):
        wid = lax.axis_index("s") * nc + lax.axis_index("c")
        base = wid * per_w
        for c in range(nchunk):
            off = base + c * ch
            pltpu.sync_copy(w_hbm.at[pl.ds(off, ch)], buf)
            pltpu.sync_copy(buf, o_hbm.at[pl.ds(off, ch)])

    return ck(flat)


def kernel(x, enc_w1, enc_b1, enc_w2, enc_b2, dec_w1, dec_b1, dec_w2, dec_b2):
    n, lp1 = x.shape
    l = lp1 - 1
    e = dec_w1.shape[0]
    nsteps = n // _BLK + e

    ids = x[:, l].astype(jnp.int32)
    inv, sched = _dispatch_plan(ids, e, n, nsteps)

    z = _encoder_call(x, enc_w1, enc_b1.reshape(1, -1),
                      enc_w2, enc_b2.reshape(1, -1))
    zs = _scatter_rows(z, inv)
    probe = _sc_copy_probe(dec_w1)
    out = _decoder_call(zs, dec_w1, dec_b1.reshape(e, 1, -1),
                        dec_w2, dec_b2.reshape(e, 1, -1), sched, nsteps)
    return out + probe[0] * 0.0


# speedup vs baseline: 1.1938x; 1.1938x over previous
import functools

import jax
import jax.numpy as jnp
from jax import lax
from jax.experimental import pallas as pl
from jax.experimental.pallas import tpu as pltpu
from jax.experimental.pallas import tpu_sc as plsc

_BLK = 256


def _leaky(v):
    return jnp.where(v >= 0, v, 0.01 * v)


def _encoder_call(x, w1, b1, w2, b2):
    n = x.shape[0]
    l = x.shape[1] - 1
    inter = w1.shape[0]
    enc = w2.shape[0]
    nb = n // _BLK

    def body(x_ref, w1_ref, b1_ref, w2_ref, b2_ref, z_ref, w1b, w2b):
        @pl.when(pl.program_id(0) == 0)
        def _():
            w1b[...] = w1_ref[...].astype(jnp.bfloat16)
            w2b[...] = w2_ref[...].astype(jnp.bfloat16)

        xb = x_ref[...].astype(jnp.bfloat16)
        h = lax.dot_general(xb, w1b[...], (((1,), (1,)), ((), ())),
                            preferred_element_type=jnp.float32)
        h = _leaky(h + b1_ref[...]).astype(jnp.bfloat16)
        z = lax.dot_general(h, w2b[...], (((1,), (1,)), ((), ())),
                            preferred_element_type=jnp.float32)
        z_ref[...] = _leaky(z + b2_ref[...])

    return pl.pallas_call(
        body,
        grid=(nb,),
        in_specs=[
            pl.BlockSpec((_BLK, l), lambda i: (i, 0)),
            pl.BlockSpec((inter, l), lambda i: (0, 0)),
            pl.BlockSpec((1, inter), lambda i: (0, 0)),
            pl.BlockSpec((enc, inter), lambda i: (0, 0)),
            pl.BlockSpec((1, enc), lambda i: (0, 0)),
        ],
        out_specs=pl.BlockSpec((_BLK, enc), lambda i: (i, 0)),
        out_shape=jax.ShapeDtypeStruct((n, enc), jnp.float32),
        scratch_shapes=[
            pltpu.VMEM((inter, l), jnp.bfloat16),
            pltpu.VMEM((enc, inter), jnp.bfloat16),
        ],
    )(x, w1, b1, w2, b2)


def _scatter_rows(z, inv):
    n, enc = z.shape
    info = plsc.get_sparse_core_info()
    nc = info.num_cores
    nw = nc * info.num_subcores
    rows_per_w = n // nw
    ch = min(rows_per_w, 128)
    nchunk = rows_per_w // ch
    mesh = plsc.VectorSubcoreMesh(core_axis_name="c", subcore_axis_name="s")

    @functools.partial(
        pl.kernel, mesh=mesh,
        out_type=jax.ShapeDtypeStruct((n, enc), jnp.float32),
        scratch_types=[
            pltpu.VMEM((ch,), jnp.int32),
            pltpu.VMEM((ch, enc), jnp.float32),
            pltpu.SemaphoreType.DMA,
        ],
    )
    def gk(z_hbm, inv_hbm, out_hbm, idx_v, rows_v, sem):
        wid = lax.axis_index("s") * nc + lax.axis_index("c")
        base = wid * rows_per_w
        for c in range(nchunk):
            off = base + c * ch
            pltpu.sync_copy(inv_hbm.at[pl.ds(off, ch)], idx_v)
            pltpu.sync_copy(z_hbm.at[pl.ds(off, ch)], rows_v)
            pltpu.async_copy(rows_v, out_hbm.at[idx_v], sem).wait()

    return gk(z, inv)


def _decoder_call(zs, w1, b1, w2, b2, sched, nsteps):
    n, enc = zs.shape
    inter = w1.shape[1]
    l = w2.shape[1]

    def body(sched_ref, zs_ref, w1_ref, b1_ref, w2_ref, b2_ref, out_ref):
        j = pl.program_id(0)
        lo = sched_ref[2, j]
        hi = sched_ref[3, j]
        h = lax.dot_general(zs_ref[...], w1_ref[0], (((1,), (1,)), ((), ())),
                            preferred_element_type=jnp.float32)
        h = _leaky(h + b1_ref[0])
        o = lax.dot_general(h, w2_ref[0], (((1,), (1,)), ((), ())),
                            preferred_element_type=jnp.float32)
        o = o + b2_ref[0]
        rows = lax.broadcasted_iota(jnp.int32, (_BLK, 1), 0)
        mask = (rows >= lo) & (rows < hi)
        out_ref[...] = jnp.where(mask, o, out_ref[...])

    grid_spec = pltpu.PrefetchScalarGridSpec(
        num_scalar_prefetch=1,
        grid=(nsteps,),
        in_specs=[
            pl.BlockSpec((_BLK, enc), lambda j, s: (s[1, j], 0)),
            pl.BlockSpec((1, inter, enc), lambda j, s: (s[0, j], 0, 0)),
            pl.BlockSpec((1, 1, inter), lambda j, s: (s[0, j], 0, 0)),
            pl.BlockSpec((1, l, inter), lambda j, s: (s[0, j], 0, 0)),
            pl.BlockSpec((1, 1, l), lambda j, s: (s[0, j], 0, 0)),
        ],
        out_specs=pl.BlockSpec((_BLK, l), lambda j, s: (s[1, j], 0)),
    )
    return pl.pallas_call(
        body,
        grid_spec=grid_spec,
        out_shape=jax.ShapeDtypeStruct((n, l), jnp.float32),
    )(sched, zs, w1, b1, w2, b2)


def _dispatch_plan(ids, e, n, nsteps):
    nb = n // _BLK
    oh = (ids[None, :] == jnp.arange(e, dtype=jnp.int32)[:, None])
    cum = jnp.cumsum(oh.astype(jnp.int32), axis=1)
    counts = cum[:, -1]
    seg_end = jnp.cumsum(counts)
    seg_start = seg_end - counts
    inv = jnp.sum(jnp.where(oh, cum + seg_start[:, None], 0), axis=0) - 1
    inv = inv.astype(jnp.int32)

    first_blk = seg_start // _BLK
    last_blk = jnp.where(counts > 0, (seg_end - 1) // _BLK, first_blk)
    steps_e = jnp.where(counts > 0, last_blk - first_blk + 1, 0)
    cum_steps = jnp.cumsum(steps_e)
    off_e = cum_steps - steps_e
    total = cum_steps[-1]

    jj = jnp.arange(nsteps, dtype=jnp.int32)
    e_j = jnp.sum(jj[:, None] >= cum_steps[None, :], axis=1).astype(jnp.int32)
    e_j = jnp.minimum(e_j, e - 1)
    blk_j = first_blk[e_j] + (jj - off_e[e_j])
    lo = jnp.maximum(seg_start[e_j] - blk_j * _BLK, 0)
    hi = jnp.minimum(seg_end[e_j] - blk_j * _BLK, _BLK)

    dummy = jj >= total
    e_last = jnp.max(jnp.where(counts > 0, jnp.arange(e, dtype=jnp.int32), -1))
    e_j = jnp.where(dummy, e_last, e_j)
    blk_j = jnp.where(dummy, nb - 1, blk_j)
    lo = jnp.where(dummy, 0, lo)
    hi = jnp.where(dummy, 0, hi)
    sched = jnp.stack([e_j, blk_j, lo, hi]).astype(jnp.int32)
    return inv, sched


def kernel(x, enc_w1, enc_b1, enc_w2, enc_b2, dec_w1, dec_b1, dec_w2, dec_b2):
    n, lp1 = x.shape
    l = lp1 - 1
    e = dec_w1.shape[0]
    nsteps = n // _BLK + e

    ids = x[:, l].astype(jnp.int32)
    inv, sched = _dispatch_plan(ids, e, n, nsteps)

    z = _encoder_call(x, enc_w1, enc_b1.reshape(1, -1),
                      enc_w2, enc_b2.reshape(1, -1))
    zs = _scatter_rows(z, inv)
    return _decoder_call(zs, dec_w1, dec_b1.reshape(e, 1, -1),
                         dec_w2, dec_b2.reshape(e, 1, -1), sched, nsteps)


# speedup vs baseline: 1.2269x; 1.0277x over previous
import functools

import jax
import jax.numpy as jnp
from jax import lax
from jax.experimental import pallas as pl
from jax.experimental.pallas import tpu as pltpu
from jax.experimental.pallas import tpu_sc as plsc

_EBLK = 512
_BLK = 256


def _leaky(v):
    return jnp.where(v >= 0, v, 0.01 * v)


def _encoder_call(x, w1, b1, w2, b2):
    n = x.shape[0]
    l = x.shape[1] - 1
    inter = w1.shape[0]
    enc = w2.shape[0]
    nb = n // _EBLK

    def body(x_ref, w1_ref, b1_ref, w2_ref, b2_ref, z_ref, w1b, w2b):
        @pl.when(pl.program_id(0) == 0)
        def _():
            w1b[...] = w1_ref[...].astype(jnp.bfloat16)
            w2b[...] = w2_ref[...].astype(jnp.bfloat16)

        xb = x_ref[...].astype(jnp.bfloat16)
        h = lax.dot_general(xb, w1b[...], (((1,), (1,)), ((), ())),
                            preferred_element_type=jnp.float32)
        h = _leaky(h + b1_ref[...]).astype(jnp.bfloat16)
        z = lax.dot_general(h, w2b[...], (((1,), (1,)), ((), ())),
                            preferred_element_type=jnp.float32)
        z_ref[...] = _leaky(z + b2_ref[...])

    return pl.pallas_call(
        body,
        grid=(nb,),
        in_specs=[
            pl.BlockSpec((_EBLK, l), lambda i: (i, 0)),
            pl.BlockSpec((inter, l), lambda i: (0, 0)),
            pl.BlockSpec((1, inter), lambda i: (0, 0)),
            pl.BlockSpec((enc, inter), lambda i: (0, 0)),
            pl.BlockSpec((1, enc), lambda i: (0, 0)),
        ],
        out_specs=pl.BlockSpec((_EBLK, enc), lambda i: (i, 0)),
        out_shape=jax.ShapeDtypeStruct((n, enc), jnp.float32),
        scratch_shapes=[
            pltpu.VMEM((inter, l), jnp.bfloat16),
            pltpu.VMEM((enc, inter), jnp.bfloat16),
        ],
    )(x, w1, b1, w2, b2)


def _scatter_rows(z, inv):
    n, enc = z.shape
    info = plsc.get_sparse_core_info()
    nc = info.num_cores
    nw = nc * info.num_subcores
    rows_per_w = n // nw
    ch = min(rows_per_w, 128)
    nchunk = rows_per_w // ch
    mesh = plsc.VectorSubcoreMesh(core_axis_name="c", subcore_axis_name="s")

    @functools.partial(
        pl.kernel, mesh=mesh,
        out_type=jax.ShapeDtypeStruct((n, enc), jnp.float32),
        scratch_types=[
            pltpu.VMEM((ch,), jnp.int32),
            pltpu.VMEM((ch, enc), jnp.float32),
            pltpu.SemaphoreType.DMA,
        ],
    )
    def gk(z_hbm, inv_hbm, out_hbm, idx_v, rows_v, sem):
        wid = lax.axis_index("s") * nc + lax.axis_index("c")
        base = wid * rows_per_w
        for c in range(nchunk):
            off = base + c * ch
            pltpu.sync_copy(inv_hbm.at[pl.ds(off, ch)], idx_v)
            pltpu.sync_copy(z_hbm.at[pl.ds(off, ch)], rows_v)
            pltpu.async_copy(rows_v, out_hbm.at[idx_v], sem).wait()

    return gk(z, inv)


def _decoder_call(zs, w1, b1, w2, b2, sched, nsteps):
    n, enc = zs.shape
    inter = w1.shape[1]
    l = w2.shape[1]

    def body(sched_ref, zs_ref, w1_ref, b1_ref, w2_ref, b2_ref, out_ref):
        j = pl.program_id(0)
        lo = sched_ref[2, j]
        hi = sched_ref[3, j]
        h = lax.dot_general(zs_ref[...], w1_ref[0], (((1,), (1,)), ((), ())),
                            preferred_element_type=jnp.float32)
        h = _leaky(h + b1_ref[0])
        o = lax.dot_general(h, w2_ref[0], (((1,), (1,)), ((), ())),
                            preferred_element_type=jnp.float32)
        o = o + b2_ref[0]
        rows = lax.broadcasted_iota(jnp.int32, (_BLK, 1), 0)
        mask = (rows >= lo) & (rows < hi)
        out_ref[...] = jnp.where(mask, o, out_ref[...])

    grid_spec = pltpu.PrefetchScalarGridSpec(
        num_scalar_prefetch=1,
        grid=(nsteps,),
        in_specs=[
            pl.BlockSpec((_BLK, enc), lambda j, s: (s[1, j], 0)),
            pl.BlockSpec((1, inter, enc), lambda j, s: (s[0, j], 0, 0)),
            pl.BlockSpec((1, 1, inter), lambda j, s: (s[0, j], 0, 0)),
            pl.BlockSpec((1, l, inter), lambda j, s: (s[0, j], 0, 0)),
            pl.BlockSpec((1, 1, l), lambda j, s: (s[0, j], 0, 0)),
        ],
        out_specs=pl.BlockSpec((_BLK, l), lambda j, s: (s[1, j], 0)),
    )
    return pl.pallas_call(
        body,
        grid_spec=grid_spec,
        out_shape=jax.ShapeDtypeStruct((n, l), jnp.float32),
    )(sched, zs, w1, b1, w2, b2)


def _dispatch_plan(ids, e, n, nsteps):
    nb = n // _BLK
    oh = (ids[None, :] == jnp.arange(e, dtype=jnp.int32)[:, None])
    cum = jnp.cumsum(oh.astype(jnp.int32), axis=1)
    counts = cum[:, -1]
    seg_end = jnp.cumsum(counts)
    seg_start = seg_end - counts
    inv = jnp.sum(jnp.where(oh, cum + seg_start[:, None], 0), axis=0) - 1
    inv = inv.astype(jnp.int32)

    first_blk = seg_start // _BLK
    last_blk = jnp.where(counts > 0, (seg_end - 1) // _BLK, first_blk)
    steps_e = jnp.where(counts > 0, last_blk - first_blk + 1, 0)
    cum_steps = jnp.cumsum(steps_e)
    off_e = cum_steps - steps_e
    total = cum_steps[-1]

    jj = jnp.arange(nsteps, dtype=jnp.int32)
    e_j = jnp.sum(jj[:, None] >= cum_steps[None, :], axis=1).astype(jnp.int32)
    e_j = jnp.minimum(e_j, e - 1)
    blk_j = first_blk[e_j] + (jj - off_e[e_j])
    lo = jnp.maximum(seg_start[e_j] - blk_j * _BLK, 0)
    hi = jnp.minimum(seg_end[e_j] - blk_j * _BLK, _BLK)

    dummy = jj >= total
    e_last = jnp.max(jnp.where(counts > 0, jnp.arange(e, dtype=jnp.int32), -1))
    e_j = jnp.where(dummy, e_last, e_j)
    blk_j = jnp.where(dummy, nb - 1, blk_j)
    lo = jnp.where(dummy, 0, lo)
    hi = jnp.where(dummy, 0, hi)
    sched = jnp.stack([e_j, blk_j, lo, hi]).astype(jnp.int32)
    return inv, sched


def kernel(x, enc_w1, enc_b1, enc_w2, enc_b2, dec_w1, dec_b1, dec_w2, dec_b2):
    n, lp1 = x.shape
    l = lp1 - 1
    e = dec_w1.shape[0]
    nsteps = n // _BLK + e

    ids = x[:, l].astype(jnp.int32)
    inv, sched = _dispatch_plan(ids, e, n, nsteps)

    z = _encoder_call(x, enc_w1, enc_b1.reshape(1, -1),
                      enc_w2, enc_b2.reshape(1, -1))
    zs = _scatter_rows(z, inv)
    return _decoder_call(zs, dec_w1, dec_b1.reshape(e, 1, -1),
                         dec_w2, dec_b2.reshape(e, 1, -1), sched, nsteps)
